# baseline (speedup 1.0000x reference)
import jax
import jax.numpy as jnp
from jax.experimental import pallas as pl
from jax.experimental.pallas import tpu as pltpu

_BLOCK = 1024
_KSPLIT = 2


def _router_kernel(x_ref, w_ref, o_ref, acc_ref):
    j = pl.program_id(1)
    khalf = x_ref.shape[1]

    @pl.when(j == 0)
    def _():
        acc_ref[...] = jax.lax.dot_general(
            w_ref[:, :khalf],
            x_ref[...],
            dimension_numbers=(((1,), (1,)), ((), ())),
            preferred_element_type=jnp.float32,
        )

    @pl.when(j == 1)
    def _():
        logits = acc_ref[...] + jax.lax.dot_general(
            w_ref[:, khalf:],
            x_ref[...],
            dimension_numbers=(((1,), (1,)), ((), ())),
            preferred_element_type=jnp.float32,
        )
        m = jnp.max(logits, axis=0, keepdims=True)
        e = jnp.exp(logits - m)
        o_ref[...] = e / jnp.sum(e, axis=0, keepdims=True)


def kernel(x, W):
    n_tokens, in_dim = x.shape
    n_experts = W.shape[0]
    khalf = in_dim // _KSPLIT
    out_t = pl.pallas_call(
        _router_kernel,
        grid=(n_tokens // _BLOCK, _KSPLIT),
        in_specs=[
            pl.BlockSpec((_BLOCK, khalf), lambda i, j: (i, j)),
            pl.BlockSpec((n_experts, in_dim), lambda i, j: (0, 0)),
        ],
        out_specs=pl.BlockSpec((n_experts, _BLOCK), lambda i, j: (0, i)),
        out_shape=jax.ShapeDtypeStruct((n_experts, n_tokens), jnp.float32),
        scratch_shapes=[pltpu.VMEM((n_experts, _BLOCK), jnp.float32)],
        compiler_params=pltpu.CompilerParams(
            dimension_semantics=("arbitrary", "arbitrary")
        ),
    )(x, W)
    return out_t.T
